# baseline (speedup 1.0000x reference)
import functools

import jax
import jax.numpy as jnp
from jax import lax
from jax.experimental import pallas as pl
from jax.experimental.pallas import tpu as pltpu, tpu_sc as plsc

B = 16384
D = 64
NC = 2
NS = 16
NW = NC * NS
BPW = B // NW
CH = 128
NCHUNK = BPW // CH


def _sc_kernel(user_tab, item_tab, uids2d, iids2d, out_hbm,
               idx_u, idx_i, half_u, half_i, u_rows, i_rows, out_v, sem):
    wid = lax.axis_index("s") * NC + lax.axis_index("c")

    pltpu.sync_copy(uids2d.at[pl.ds(wid * NCHUNK, NCHUNK)], idx_u)
    pltpu.sync_copy(iids2d.at[pl.ds(wid * NCHUNK, NCHUNK)], idx_i)

    for r in range(NCHUNK):
        for k in range(CH // 16):
            sl = pl.ds(k * 16, 16)
            half_u[r, sl] = lax.shift_right_logical(idx_u[r, sl], 1)
            half_i[r, sl] = lax.shift_right_logical(idx_i[r, sl], 1)

    def fire(c, buf):
        return (pltpu.async_copy(user_tab.at[half_u.at[c]], u_rows.at[buf], sem),
                pltpu.async_copy(item_tab.at[half_i.at[c]], i_rows.at[buf], sem))

    lane = lax.iota(jnp.int32, 16)
    perms = [(lane + sh) & 15 for sh in (8, 4, 2, 1)]

    def compute_chunk(c, buf):
        def group(g, carry):
            acc = jnp.zeros((16,), jnp.float32)
            uo_vec = (idx_u[c, pl.ds(g * 16, 16)] & 1) * 64
            io_vec = (idx_i[c, pl.ds(g * 16, 16)] & 1) * 64
            for t in range(16):
                b = g * 16 + t
                uo = uo_vec[t]
                io = io_vec[t]
                p = (u_rows[buf, b, pl.ds(uo, 16)] *
                     i_rows[buf, b, pl.ds(io, 16)])
                for j in range(1, D // 16):
                    p += (u_rows[buf, b, pl.ds(uo + j * 16, 16)] *
                          i_rows[buf, b, pl.ds(io + j * 16, 16)])
                for perm in perms:
                    p = p + p.at[perm].get(mode="promise_in_bounds")
                acc = jnp.where(lane == t, p, acc)
            out_v[pl.ds(c * CH + g * 16, 16)] = acc
            return carry
        lax.fori_loop(0, CH // 16, group, 0)

    pending = fire(0, 0)
    for c in range(NCHUNK):
        for cp in pending:
            cp.wait()
        if c + 1 < NCHUNK:
            pending = fire(c + 1, (c + 1) % 2)
        compute_chunk(c, c % 2)

    pltpu.sync_copy(out_v, out_hbm.at[pl.ds(wid * BPW, BPW)])


@jax.jit
def kernel(user_ids, item_ids, user_table, item_table):
    ut2 = user_table.reshape(500000, 128)
    it2 = item_table.reshape(500000, 128)
    uids2d = user_ids.reshape(NW * NCHUNK, CH)
    iids2d = item_ids.reshape(NW * NCHUNK, CH)
    mesh = plsc.VectorSubcoreMesh(core_axis_name="c", subcore_axis_name="s")
    run = functools.partial(
        pl.kernel, mesh=mesh,
        out_type=jax.ShapeDtypeStruct((B,), jnp.float32),
        scratch_types=[
            pltpu.VMEM((NCHUNK, CH), jnp.int32),
            pltpu.VMEM((NCHUNK, CH), jnp.int32),
            pltpu.VMEM((NCHUNK, CH), jnp.int32),
            pltpu.VMEM((NCHUNK, CH), jnp.int32),
            pltpu.VMEM((2, CH, 128), jnp.float32),
            pltpu.VMEM((2, CH, 128), jnp.float32),
            pltpu.VMEM((BPW,), jnp.float32),
            pltpu.SemaphoreType.DMA,
        ],
    )(_sc_kernel)
    return run(ut2, it2, uids2d, iids2d)


# speedup vs baseline: 2.3494x; 2.3494x over previous
import functools

import jax
import jax.numpy as jnp
from jax import lax
from jax.experimental import pallas as pl
from jax.experimental.pallas import tpu as pltpu, tpu_sc as plsc

B = 16384
D = 64
NC = 2
NS = 16
NW = NC * NS
NCOL = 7813
CPW = 245
MCAP = 768
NCHK = MCAP // 128


def _stream_pass(is_item):
    def body(*refs):
        if is_item:
            (tabr, idsr, staging, out, ids_v, cb0, cb1, mids, mbuf, mbs2d,
             colmatch, ebuf, ubuf, wide, sem0, sem1, semg) = refs
        else:
            (tabr, idsr, staging, ids_v, cb0, cb1, mids, mbuf, mbs2d,
             colmatch, ebuf, wide, sem0, sem1, semg) = refs

        wid = lax.axis_index("s") * NC + lax.axis_index("c")
        ncols = jnp.where(wid < NW - 1, CPW, NCOL - (NW - 1) * CPW)
        wstart = wid * CPW
        lo = wstart * 128
        hi = lo + ncols * 128
        lane = lax.iota(jnp.int32, 16)
        zero16 = jnp.zeros((16,), jnp.int32)

        pltpu.sync_copy(idsr, ids_v)

        neg1 = jnp.full((16,), -1, jnp.int32)
        for r in range(NCHK):
            for k in range(8):
                mbs2d[r, pl.ds(k * 16, 16)] = neg1

        def scan_step(i, cnt):
            r = i // 8
            k = i % 8
            idv = ids_v[r, pl.ds(k * 16, 16)]
            m = (idv >= lo) & (idv < hi)
            plsc.store_compressed(mids.at[pl.ds(cnt, 16)], idv, mask=m)
            plsc.store_compressed(mbuf.at[pl.ds(cnt, 16)], i * 16 + lane,
                                  mask=m)
            cnt = cnt + plsc.all_reduce_population_count(m)[0]
            return jnp.minimum(cnt, MCAP - 16)

        cnt = lax.fori_loop(0, B // 16, scan_step, jnp.int32(0), unroll=8)
        cnt16 = (cnt + 15) // 16

        def fire(c, buf, sem):
            return pltpu.async_copy(
                tabr.at[:, pl.ds((wstart + c) * 128, 128)], buf, sem)

        @pl.when(0 < ncols)
        def _():
            fire(0, cb0, sem0)

        @pl.when(1 < ncols)
        def _():
            fire(1, cb1, sem1)

        def process_col(c, buf, sem, cm):
            pltpu.make_async_copy(
                tabr.at[:, pl.ds(0, 128)], buf, sem).wait()

            col_abs = wstart + c

            def rescan(j, cc):
                idv = mids[pl.ds(j * 16, 16)]
                mcol = ((idv >> 7) == col_abs) & (j * 16 + lane < cnt)
                plsc.store_compressed(colmatch.at[pl.ds(cc, 16)],
                                      j * 16 + lane, mask=mcol)
                return cc + plsc.all_reduce_population_count(mcol)[0]

            ccount = lax.fori_loop(0, cnt16, rescan, jnp.int32(0))

            def extract(s, cm_in):
                pos = plsc.load_gather(colmatch, [jnp.broadcast_to(s, (16,))])
                idv = plsc.load_gather(mids, [pos])
                bv = plsc.load_gather(mbuf, [pos])
                lane_in_col = idv & 127
                cm_safe = jnp.minimum(cm_in, MCAP - 1)
                erow = cm_safe >> 1
                eoff = (cm_safe & 1) * 64
                for j in range(D // 16):
                    vals = plsc.load_gather(
                        buf, [j * 16 + lane, lane_in_col])
                    ebuf[erow, pl.ds(eoff + j * 16, 16)] = vals
                plsc.store_scatter(
                    mbs2d,
                    [jnp.broadcast_to(cm_safe >> 7, (16,)),
                     jnp.broadcast_to(cm_safe & 127, (16,))],
                    bv, mask=lane == 0)
                return cm_in + 1

            return lax.fori_loop(0, ccount, extract, cm)

        def superstep(s, cm):
            c0 = s * 2
            c1 = s * 2 + 1
            cm = lax.cond(c0 < ncols,
                          lambda x: process_col(c0, cb0, sem0, x),
                          lambda x: x, cm)

            @pl.when(c0 + 2 < ncols)
            def _():
                fire(c0 + 2, cb0, sem0)

            cm = lax.cond(c1 < ncols,
                          lambda x: process_col(c1, cb1, sem1, x),
                          lambda x: x, cm)

            @pl.when(c1 + 2 < ncols)
            def _():
                fire(c1 + 2, cb1, sem1)

            return cm

        lax.fori_loop(0, (CPW + 1) // 2, superstep, jnp.int32(0))

        if not is_item:
            for k in range(NCHK):
                def widen(li, carry, k=k):
                    m = k * 128 + li
                    for j in range(D // 16):
                        wide[li, pl.ds(j * 16, 16)] = \
                            ebuf[m >> 1, pl.ds((m & 1) * 64 + j * 16, 16)]
                    return carry

                lax.fori_loop(0, 128, widen, 0)
                pltpu.async_copy(
                    wide,
                    staging.at[plsc.Indices(mbs2d.at[k], ignored_value=-1)],
                    semg).wait()
        else:
            perms = [(lane + sh) & 15 for sh in (8, 4, 2, 1)]
            for k in range(NCHK):
                pltpu.async_copy(
                    staging.at[plsc.Indices(mbs2d.at[k], ignored_value=-1)],
                    ubuf, semg).wait()

                def group(g, carry, k=k):
                    acc = jnp.zeros((16,), jnp.float32)
                    for t in range(16):
                        li = g * 16 + t
                        erow = (k * 128 + li) >> 1
                        eoff = (t & 1) * 64
                        p = (ebuf[erow, pl.ds(eoff, 16)] *
                             ubuf[li, pl.ds(0, 16)])
                        for j in range(1, D // 16):
                            p += (ebuf[erow, pl.ds(eoff + j * 16, 16)] *
                                  ubuf[li, pl.ds(j * 16, 16)])
                        for perm in perms:
                            p = p + p.at[perm].get(mode="promise_in_bounds")
                        acc = jnp.where(lane == t, p, acc)
                    plsc.store_scatter(wide, [g * 16 + lane, zero16], acc)
                    return carry

                lax.fori_loop(0, 8, group, 0)
                pltpu.async_copy(
                    wide,
                    out.at[plsc.Indices(mbs2d.at[k], ignored_value=-1)],
                    semg).wait()

    return body


def _make_kernel(is_item):
    mesh = plsc.VectorSubcoreMesh(core_axis_name="c", subcore_axis_name="s")
    scratch = [
        pltpu.VMEM((B // 128, 128), jnp.int32),
        pltpu.VMEM((D, 128), jnp.float32),
        pltpu.VMEM((D, 128), jnp.float32),
        pltpu.VMEM((MCAP,), jnp.int32),
        pltpu.VMEM((MCAP,), jnp.int32),
        pltpu.VMEM((NCHK, 128), jnp.int32),
        pltpu.VMEM((MCAP,), jnp.int32),
        pltpu.VMEM((MCAP // 2, 2 * D), jnp.float32),
    ]
    if is_item:
        scratch += [pltpu.VMEM((128, 128), jnp.float32)]
    scratch += [pltpu.VMEM((128, 128), jnp.float32)]
    scratch += [pltpu.SemaphoreType.DMA, pltpu.SemaphoreType.DMA,
                pltpu.SemaphoreType.DMA]
    out_type = jax.ShapeDtypeStruct((B, 128), jnp.float32)
    return functools.partial(
        pl.kernel, mesh=mesh, out_type=out_type, scratch_types=scratch,
        compiler_params=pltpu.CompilerParams(needs_layout_passes=False),
    )(_stream_pass(is_item))


@jax.jit
def kernel(user_ids, item_ids, user_table, item_table):
    ut_t = user_table.T
    it_t = item_table.T
    uids2d = user_ids.reshape(B // 128, 128)
    iids2d = item_ids.reshape(B // 128, 128)
    staging = _make_kernel(False)(ut_t, uids2d)
    out2d = _make_kernel(True)(it_t, iids2d, staging)
    return out2d[:, 0]
